# baseline (speedup 1.0000x reference)
import math

import jax
import jax.numpy as jnp
from jax import lax
from jax.experimental import pallas as pl
from jax.experimental.pallas import tpu as pltpu
from jax.experimental.pallas import tpu_sc as plsc

TAIL = 8


def _sc_gather_body(x_hbm, table_hbm, out_hbm, idx_v, rows_v, row1_v, sem):
    wid = lax.axis_index("s") * 2 + lax.axis_index("c")

    @pl.when(wid == 0)
    def _():
        L = x_hbm.shape[0]
        pltpu.sync_copy(x_hbm.at[pl.ds(L - TAIL, TAIL)], idx_v)
        pltpu.async_copy(table_hbm.at[idx_v], rows_v, sem).wait()
        pltpu.sync_copy(table_hbm.at[pl.ds(1, 1)], row1_v)
        pltpu.sync_copy(row1_v, out_hbm.at[pl.ds(0, 1)])
        pltpu.sync_copy(rows_v.at[pl.ds(TAIL - 1, 1)], out_hbm.at[pl.ds(1, 1)])


def _tc_pos_add_body(rows_ref, out_ref):
    d = lax.broadcasted_iota(jnp.int32, out_ref.shape, 1).astype(jnp.float32)
    out_ref[...] = rows_ref[...] + jnp.sin(d * (math.pi / 180.0))


def kernel(x, embed_table):
    d_model = embed_table.shape[1]
    mesh = plsc.VectorSubcoreMesh(core_axis_name="c", subcore_axis_name="s")
    rows = pl.kernel(
        _sc_gather_body,
        out_type=jax.ShapeDtypeStruct((2, d_model), jnp.float32),
        mesh=mesh,
        scratch_types=[
            pltpu.VMEM((TAIL,), jnp.int32),
            pltpu.VMEM((TAIL, d_model), jnp.float32),
            pltpu.VMEM((1, d_model), jnp.float32),
            pltpu.SemaphoreType.DMA,
        ],
    )(x, embed_table)

    return pl.pallas_call(
        _tc_pos_add_body,
        out_shape=jax.ShapeDtypeStruct((2, d_model), jnp.float32),
    )(rows)


# speedup vs baseline: 1.0949x; 1.0949x over previous
import math

import numpy as np
import jax
import jax.numpy as jnp
from jax import lax
from jax.experimental import pallas as pl
from jax.experimental.pallas import tpu as pltpu
from jax.experimental.pallas import tpu_sc as plsc

LANES = 16
D_MODEL = 512

_POS_ROW = np.sin(np.arange(D_MODEL, dtype=np.float64) * (math.pi / 180.0)).astype(
    np.float32
)


def _sc_body(x_hbm, pos_hbm, table_hbm, out_hbm, xt_v, idx_v, rows_v, pos_v, sems):
    wid = lax.axis_index("s") * 2 + lax.axis_index("c")

    @pl.when(wid == 0)
    def _():
        L = x_hbm.shape[0]
        tail_cp = pltpu.async_copy(x_hbm.at[pl.ds(L - LANES, LANES)], xt_v, sems.at[0])
        pos_cp = pltpu.async_copy(pos_hbm, pos_v, sems.at[1])
        tail_cp.wait()
        lane = lax.iota(jnp.int32, LANES)
        idx_v[...] = jnp.where(lane == LANES - 2, 1, xt_v[...])
        pltpu.async_copy(table_hbm.at[idx_v], rows_v, sems.at[2]).wait()
        pos_cp.wait()
        for r in range(LANES - 2, LANES):
            for c in range(D_MODEL // LANES):
                sl = pl.ds(LANES * c, LANES)
                rows_v[r, sl] += pos_v[sl]
        pltpu.sync_copy(rows_v.at[pl.ds(LANES - 2, 2)], out_hbm)


def kernel(x, embed_table):
    mesh = plsc.VectorSubcoreMesh(core_axis_name="c", subcore_axis_name="s")
    return pl.kernel(
        _sc_body,
        out_type=jax.ShapeDtypeStruct((2, D_MODEL), jnp.float32),
        mesh=mesh,
        scratch_types=[
            pltpu.VMEM((LANES,), jnp.int32),
            pltpu.VMEM((LANES,), jnp.int32),
            pltpu.VMEM((LANES, D_MODEL), jnp.float32),
            pltpu.VMEM((D_MODEL,), jnp.float32),
            pltpu.SemaphoreType.DMA((3,)),
        ],
    )(x, jnp.asarray(_POS_ROW), embed_table)


# speedup vs baseline: 1.1811x; 1.0787x over previous
import math

import numpy as np
import jax
import jax.numpy as jnp
from jax import lax
from jax.experimental import pallas as pl
from jax.experimental.pallas import tpu as pltpu
from jax.experimental.pallas import tpu_sc as plsc

LANES = 16
D_MODEL = 512

_POS_ROW = np.sin(np.arange(D_MODEL, dtype=np.float64) * (math.pi / 180.0)).astype(
    np.float32
)


def _sc_body(x_hbm, pos_hbm, table_hbm, out_hbm, xt_v, idx_v, rows_v, pos_v, sems):
    wid = lax.axis_index("s") * 2 + lax.axis_index("c")

    @pl.when(wid == 0)
    def _():
        L = x_hbm.shape[0]
        tail_cp = pltpu.async_copy(x_hbm.at[pl.ds(L - LANES, LANES)], xt_v, sems.at[0])
        pos_cp = pltpu.async_copy(pos_hbm, pos_v, sems.at[1])
        tail_cp.wait()
        lane = lax.iota(jnp.int32, LANES)
        idx_v[...] = jnp.where(lane == LANES - 2, 1, xt_v[...])
        pltpu.async_copy(table_hbm.at[idx_v], rows_v, sems.at[2]).wait()
        pos_cp.wait()
        for r in range(LANES - 2, LANES):
            for c in range(D_MODEL // LANES):
                sl = pl.ds(LANES * c, LANES)
                rows_v[r, sl] += pos_v[sl]
        pltpu.sync_copy(rows_v.at[pl.ds(LANES - 2, 2)], out_hbm)


def kernel(x, embed_table):
    mesh = plsc.VectorSubcoreMesh(
        core_axis_name="c", subcore_axis_name="s", num_cores=1
    )
    return pl.kernel(
        _sc_body,
        out_type=jax.ShapeDtypeStruct((2, D_MODEL), jnp.float32),
        mesh=mesh,
        scratch_types=[
            pltpu.VMEM((LANES,), jnp.int32),
            pltpu.VMEM((LANES,), jnp.int32),
            pltpu.VMEM((LANES, D_MODEL), jnp.float32),
            pltpu.VMEM((D_MODEL,), jnp.float32),
            pltpu.SemaphoreType.DMA((3,)),
        ],
    )(x, jnp.asarray(_POS_ROW), embed_table)


# speedup vs baseline: 1.1841x; 1.0026x over previous
import math

import numpy as np
import jax
import jax.numpy as jnp
from jax import lax
from jax.experimental import pallas as pl
from jax.experimental.pallas import tpu as pltpu
from jax.experimental.pallas import tpu_sc as plsc

LANES = 16
D_MODEL = 512

_POS_ROW = np.sin(np.arange(D_MODEL, dtype=np.float64) * (math.pi / 180.0)).astype(
    np.float32
)


def _sc_body(x_hbm, pos_hbm, table_hbm, out_hbm, xt_v, idx_v, rows_v, pos_v, sems):
    wid = lax.axis_index("s") * 2 + lax.axis_index("c")

    @pl.when(wid == 0)
    def _():
        L = x_hbm.shape[0]
        tail_cp = pltpu.async_copy(x_hbm.at[pl.ds(L - LANES, LANES)], xt_v, sems.at[0])
        pos_cp = pltpu.async_copy(pos_hbm, pos_v, sems.at[1])
        tail_cp.wait()
        lane = lax.iota(jnp.int32, LANES)
        idx_v[...] = jnp.where(lane == LANES - 2, 1, xt_v[...])
        pltpu.async_copy(table_hbm.at[idx_v], rows_v, sems.at[2]).wait()
        pos_cp.wait()
        for r in range(LANES - 2, LANES):
            for c in range(D_MODEL // LANES):
                sl = pl.ds(LANES * c, LANES)
                rows_v[r, sl] += pos_v[sl]
        pltpu.sync_copy(rows_v.at[pl.ds(LANES - 2, 2)], out_hbm)


def kernel(x, embed_table):
    mesh = plsc.VectorSubcoreMesh(
        core_axis_name="c", subcore_axis_name="s", num_cores=1, num_subcores=1
    )
    return pl.kernel(
        _sc_body,
        out_type=jax.ShapeDtypeStruct((2, D_MODEL), jnp.float32),
        mesh=mesh,
        scratch_types=[
            pltpu.VMEM((LANES,), jnp.int32),
            pltpu.VMEM((LANES,), jnp.int32),
            pltpu.VMEM((LANES, D_MODEL), jnp.float32),
            pltpu.VMEM((D_MODEL,), jnp.float32),
            pltpu.SemaphoreType.DMA((3,)),
        ],
    )(x, jnp.asarray(_POS_ROW), embed_table)
